# baseline (speedup 1.0000x reference)
import functools

import jax
import jax.numpy as jnp
from jax import lax
from jax.experimental import pallas as pl
from jax.experimental.pallas import tpu as pltpu
from jax.experimental.pallas import tpu_sc as plsc

_G = 8
_ROWS = 16384
_D = 128
_NC = 2
_NS = 16
_NW = _NC * _NS
_SC_ROWS = 6144
_RPW = _SC_ROWS // _NW
_CHUNK = 48
_NCHUNK = _RPW // _CHUNK
_PR = 2 * _G + 1
_UNROLL = 8
_TCB = 256
_TC_ROWS = _ROWS - _SC_ROWS

_SPLAT_DNUMS = lax.GatherDimensionNumbers(
    offset_dims=(), collapsed_slice_dims=(0,), start_index_map=(0,))


def _splat(vec, r):
    idx = jnp.full((16, 1), r, jnp.int32)
    return lax.gather(vec, idx, _SPLAT_DNUMS, (1,),
                      mode=lax.GatherScatterMode.PROMISE_IN_BOUNDS)


def _tree8(v):
    a = [v[2 * i] + v[2 * i + 1] for i in range(4)]
    b = [a[0] + a[1], a[2] + a[3]]
    return b[0] + b[1]


def _sc_body(p_hbm, t_hbm, lab_hbm, out_hbm, pbuf, tbuf, labv, part,
             psem, tsem):
    c = lax.axis_index("c")
    s = lax.axis_index("s")
    wid = s * _NC + c
    base = wid * _RPW

    pltpu.sync_copy(lab_hbm.at[pl.ds(base, _RPW)], labv.at[pl.ds(0, _RPW)])

    zero = jnp.zeros((16,), jnp.float32)
    for i in range(1, _PR):
        part[pl.ds(i * 16, 16)] = zero

    iota = lax.iota(jnp.int32, 16)
    iota_gs = iota + 16
    iota_cnt = iota + (1 + _G) * 16
    ones = jnp.full((16,), 1.0, jnp.float32)

    def start_chunk(ci, b):
        rb = base + ci * _CHUNK
        hp = pltpu.async_copy(p_hbm.at[pl.ds(rb, _CHUNK)], pbuf.at[b], psem)
        ht = pltpu.async_copy(t_hbm.at[pl.ds(rb, _CHUNK)], tbuf.at[b], tsem)
        return hp, ht

    handles = [start_chunk(0, 0), start_chunk(1, 1)]

    def cnt_body(bi, carry):
        labvec = labv[pl.ds(bi * 16, 16)]
        plsc.addupdate_scatter(part, [labvec * 16 + iota_cnt], ones)
        return carry
    lax.fori_loop(0, _RPW // 16, cnt_body, 0)

    def compute_chunk(b, ci, acc_c):
        def blk_body(bi, acc_i, _b=b, _ci=ci):
            r0 = bi * _UNROLL
            labvec = labv[pl.ds(_ci * _CHUNK + r0, 16)]
            acc_l = list(acc_i)
            for r in range(_UNROLL):
                row = r0 + r
                pv = [pbuf[_b, row, pl.ds(k * 16, 16)] for k in range(8)]
                tv = [tbuf[_b, row, pl.ds(k * 16, 16)] for k in range(8)]
                for k in range(8):
                    dd = pv[k] - tv[k]
                    acc_l[k % 4] = acc_l[k % 4] + dd * dd
                rp = _tree8(pv)
                lab_splat = _splat(labvec, r)
                plsc.addupdate_scatter(part, [lab_splat * 16 + iota_gs], rp)
            return tuple(acc_l)
        return lax.fori_loop(0, _CHUNK // _UNROLL, blk_body, acc_c)

    acc = (zero, zero, zero, zero)

    def pair_body(pi, acc_c):
        ci0 = pi * 2
        handles[0][0].wait()
        handles[0][1].wait()
        acc_c = compute_chunk(0, ci0, acc_c)

        @pl.when(ci0 + 2 < _NCHUNK)
        def _():
            start_chunk(ci0 + 2, 0)

        handles[1][0].wait()
        handles[1][1].wait()
        acc_c = compute_chunk(1, ci0 + 1, acc_c)

        @pl.when(ci0 + 3 < _NCHUNK)
        def _():
            start_chunk(ci0 + 3, 1)
        return acc_c

    acc = lax.fori_loop(0, _NCHUNK // 2, pair_body, acc)

    part[pl.ds(0, 16)] = (acc[0] + acc[1]) + (acc[2] + acc[3])
    pltpu.sync_copy(part, out_hbm.at[wid])


def _tc_body(p_ref, t_ref, lab_ref, o_ref):
    i = pl.program_id(0)

    @pl.when(i == 0)
    def _():
        for j in range(_PR):
            o_ref[j] = 0.0

    p = p_ref[...]
    t = t_ref[...]
    d = p - t
    o_ref[0] = o_ref[0] + jnp.sum(d * d)
    rs2 = jnp.sum(p.reshape(_TCB // 128, 128, 128), axis=2)
    lab2 = lab_ref[0]
    zf = jnp.zeros_like(rs2)
    for g in range(_G):
        m = lab2 == g
        o_ref[1 + g] = o_ref[1 + g] + jnp.sum(jnp.where(m, rs2, zf))
        o_ref[1 + _G + g] = o_ref[1 + _G + g] + jnp.sum(
            jnp.where(m, 1.0, 0.0))


@jax.jit
def _sc_partials(predictions, targets, labels):
    mesh = plsc.VectorSubcoreMesh(core_axis_name="c", subcore_axis_name="s")
    f = functools.partial(
        pl.kernel,
        out_type=jax.ShapeDtypeStruct((_NW, _PR * 16), jnp.float32),
        mesh=mesh,
        compiler_params=pltpu.CompilerParams(needs_layout_passes=False),
        scratch_types=[
            pltpu.VMEM((2, _CHUNK, _D), jnp.float32),
            pltpu.VMEM((2, _CHUNK, _D), jnp.float32),
            pltpu.VMEM((_RPW + 16,), jnp.int32),
            pltpu.VMEM((_PR * 16,), jnp.float32),
            pltpu.SemaphoreType.DMA,
            pltpu.SemaphoreType.DMA,
        ],
    )(_sc_body)
    return f(predictions, targets, labels)


_TC_OFF = _SC_ROWS // _TCB


@jax.jit
def _tc_partials(p_full, t_full, lab_full):
    grid = (_TC_ROWS // _TCB,)
    return pl.pallas_call(
        _tc_body,
        grid=grid,
        in_specs=[
            pl.BlockSpec((_TCB, _D), lambda i: (i + _TC_OFF, 0)),
            pl.BlockSpec((_TCB, _D), lambda i: (i + _TC_OFF, 0)),
            pl.BlockSpec((1, _TCB // 128, 128), lambda i: (i + _TC_OFF, 0, 0)),
        ],
        out_specs=pl.BlockSpec(memory_space=pltpu.SMEM),
        out_shape=jax.ShapeDtypeStruct((_PR,), jnp.float32),
    )(p_full, t_full, lab_full)


def kernel(predictions, targets, group_labels):
    labels = group_labels.astype(jnp.int32)
    parts = _sc_partials(predictions, targets, labels)
    lab3 = labels.reshape(_ROWS // _TCB, _TCB // 128, 128)
    tc = _tc_partials(predictions, targets, lab3)
    tot = jnp.sum(parts.reshape(_NW, _PR, 16), axis=(0, 2)) + tc
    sq = tot[0]
    gs = tot[1:1 + _G]
    cnt = tot[1 + _G:]
    n = predictions.shape[0] * predictions.shape[1]
    base_loss = sq / n
    gm = gs / (cnt * predictions.shape[1])
    mm = jnp.mean(gm)
    penalty = jnp.sum((gm - mm) ** 2) / (_G - 1)
    return base_loss + penalty
